# baseline (speedup 1.0000x reference)
import jax
import jax.numpy as jnp
import numpy as np
from jax.experimental import pallas as pl

_B = 256
_W = 25
_D = 25
_WD = _W * _D
_E = 128
_VOCAB_SIZES = (4, 8, 4, 4, 8, 8)
_NROWS = sum(_VOCAB_SIZES)
_OFFSETS = tuple(int(x) for x in np.cumsum((0,) + _VOCAB_SIZES[:-1]))

_B_BLK = 4


def _embed_body(idx_ref, tabT_ref, out_ref):
    tabT = tabT_ref[...]
    rows = jax.lax.broadcasted_iota(jnp.int32, (_NROWS, _WD), 0)
    for bb in range(_B_BLK):
        oh = jnp.zeros((_NROWS, _WD), dtype=jnp.float32)
        for f in range(6):
            idx_f = idx_ref[bb, f] + _OFFSETS[f]
            oh = oh + (rows == idx_f[None, :]).astype(jnp.float32)
        res = jnp.dot(tabT, oh, preferred_element_type=jnp.float32)
        for w in range(_W):
            out_ref[bb, :, w, :] = res[:, w * _D:(w + 1) * _D]


def kernel(card_counts, card_colors, card_shapes, card_selections,
           leader_rotations, follower_rotations,
           T_count, T_color, T_shape, T_sel, T_lead, T_foll):
    idx_all = jnp.stack(
        [a.reshape(_B, _WD) for a in (card_counts, card_colors, card_shapes,
                                      card_selections, leader_rotations,
                                      follower_rotations)], axis=1)

    tab = jnp.concatenate([T_count, T_color, T_shape, T_sel, T_lead, T_foll],
                          axis=0)
    row_mask = np.ones((_NROWS, 1), dtype=np.float32)
    for off in _OFFSETS:
        row_mask[off, 0] = 0.0
    tabT = (tab * jnp.asarray(row_mask)).T

    out = pl.pallas_call(
        _embed_body,
        grid=(_B // _B_BLK,),
        in_specs=[
            pl.BlockSpec((_B_BLK, 6, _WD), lambda i: (i, 0, 0)),
            pl.BlockSpec((_E, _NROWS), lambda i: (0, 0)),
        ],
        out_specs=pl.BlockSpec((_B_BLK, _E, _W, _D), lambda i: (i, 0, 0, 0)),
        out_shape=jax.ShapeDtypeStruct((_B, _E, _W, _D), jnp.float32),
    )(idx_all, tabT)
    return out


# speedup vs baseline: 9.2589x; 9.2589x over previous
import jax
import jax.numpy as jnp
import numpy as np
from jax.experimental import pallas as pl
from jax.experimental.pallas import tpu as pltpu

_B = 256
_W = 25
_D = 25
_WD = _W * _D
_E = 128
_VOCAB_SIZES = (4, 8, 4, 4, 8, 8)
_NROWS = sum(_VOCAB_SIZES)
_OFFSETS = tuple(int(x) for x in np.cumsum((0,) + _VOCAB_SIZES[:-1]))

_B_BLK = 8


def _embed_body(i0, i1, i2, i3, i4, i5, tabT_ref, out_ref):
    tabT = tabT_ref[...]
    idx_refs = (i0, i1, i2, i3, i4, i5)
    rows = jax.lax.broadcasted_iota(jnp.int32, (_NROWS, _WD), 0)
    for bb in range(_B_BLK):
        oh = jnp.zeros((_NROWS, _WD), dtype=jnp.float32)
        for f in range(6):
            idx_f = idx_refs[f][bb, 0] + _OFFSETS[f]
            oh = oh + (rows == idx_f[None, :]).astype(jnp.float32)
        out_ref[bb] = jnp.dot(tabT, oh, preferred_element_type=jnp.float32)


def kernel(card_counts, card_colors, card_shapes, card_selections,
           leader_rotations, follower_rotations,
           T_count, T_color, T_shape, T_sel, T_lead, T_foll):
    idx_arrays = [a.reshape(_B, 1, _WD) for a in
                  (card_counts, card_colors, card_shapes, card_selections,
                   leader_rotations, follower_rotations)]

    tab = jnp.concatenate([T_count, T_color, T_shape, T_sel, T_lead, T_foll],
                          axis=0)
    row_mask = np.ones((_NROWS, 1), dtype=np.float32)
    for off in _OFFSETS:
        row_mask[off, 0] = 0.0
    tabT = (tab * jnp.asarray(row_mask)).T

    idx_spec = pl.BlockSpec((_B_BLK, 1, _WD), lambda i: (i, 0, 0))
    out = pl.pallas_call(
        _embed_body,
        grid=(_B // _B_BLK,),
        in_specs=[idx_spec] * 6 + [pl.BlockSpec((_E, _NROWS), lambda i: (0, 0))],
        out_specs=pl.BlockSpec((_B_BLK, _E, _WD), lambda i: (i, 0, 0)),
        out_shape=jax.ShapeDtypeStruct((_B, _E, _WD), jnp.float32),
        compiler_params=pltpu.CompilerParams(
            dimension_semantics=("parallel",)),
    )(*idx_arrays, tabT)
    return out.reshape(_B, _E, _W, _D)


# speedup vs baseline: 9.8211x; 1.0607x over previous
import jax
import jax.numpy as jnp
import numpy as np
from jax.experimental import pallas as pl
from jax.experimental.pallas import tpu as pltpu

_B = 256
_W = 25
_D = 25
_WD = _W * _D
_E = 128
_VOCAB_SIZES = (4, 8, 4, 4, 8, 8)
_NROWS = sum(_VOCAB_SIZES)
_OFFSETS = tuple(int(x) for x in np.cumsum((0,) + _VOCAB_SIZES[:-1]))

_B_BLK = 16


def _embed_body(i0, i1, i2, i3, i4, i5, tabT_ref, out_ref):
    tabT = tabT_ref[...]
    idx_refs = (i0, i1, i2, i3, i4, i5)
    rows = jax.lax.broadcasted_iota(jnp.int32, (_NROWS, _WD), 0)
    for bb in range(_B_BLK):
        oh = jnp.zeros((_NROWS, _WD), dtype=jnp.float32)
        for f in range(6):
            idx_f = idx_refs[f][bb, 0] + _OFFSETS[f]
            oh = oh + (rows == idx_f[None, :]).astype(jnp.float32)
        out_ref[bb] = jnp.dot(tabT, oh, preferred_element_type=jnp.float32)


def kernel(card_counts, card_colors, card_shapes, card_selections,
           leader_rotations, follower_rotations,
           T_count, T_color, T_shape, T_sel, T_lead, T_foll):
    idx_arrays = [a.reshape(_B, 1, _WD) for a in
                  (card_counts, card_colors, card_shapes, card_selections,
                   leader_rotations, follower_rotations)]

    tab = jnp.concatenate([T_count, T_color, T_shape, T_sel, T_lead, T_foll],
                          axis=0)
    row_mask = np.ones((_NROWS, 1), dtype=np.float32)
    for off in _OFFSETS:
        row_mask[off, 0] = 0.0
    tabT = (tab * jnp.asarray(row_mask)).T

    idx_spec = pl.BlockSpec((_B_BLK, 1, _WD), lambda i: (i, 0, 0))
    out = pl.pallas_call(
        _embed_body,
        grid=(_B // _B_BLK,),
        in_specs=[idx_spec] * 6 + [pl.BlockSpec((_E, _NROWS), lambda i: (0, 0))],
        out_specs=pl.BlockSpec((_B_BLK, _E, _WD), lambda i: (i, 0, 0)),
        out_shape=jax.ShapeDtypeStruct((_B, _E, _WD), jnp.float32),
        compiler_params=pltpu.CompilerParams(
            dimension_semantics=("parallel",)),
    )(*idx_arrays, tabT)
    return out.reshape(_B, _E, _W, _D)


# speedup vs baseline: 9.9945x; 1.0177x over previous
import jax
import jax.numpy as jnp
import numpy as np
from jax.experimental import pallas as pl
from jax.experimental.pallas import tpu as pltpu

_B = 256
_W = 25
_D = 25
_WD = _W * _D
_E = 128
_VOCAB_SIZES = (4, 8, 4, 4, 8, 8)
_NROWS = sum(_VOCAB_SIZES)
_OFFSETS = tuple(int(x) for x in np.cumsum((0,) + _VOCAB_SIZES[:-1]))

_B_BLK = 32


def _embed_body(i0, i1, i2, i3, i4, i5, tabT_ref, out_ref):
    tabT = tabT_ref[...]
    idx_refs = (i0, i1, i2, i3, i4, i5)
    rows = jax.lax.broadcasted_iota(jnp.int32, (_NROWS, _WD), 0)
    for bb in range(_B_BLK):
        oh = jnp.zeros((_NROWS, _WD), dtype=jnp.float32)
        for f in range(6):
            idx_f = idx_refs[f][bb, 0] + _OFFSETS[f]
            oh = oh + (rows == idx_f[None, :]).astype(jnp.float32)
        out_ref[bb] = jnp.dot(tabT, oh, preferred_element_type=jnp.float32)


def kernel(card_counts, card_colors, card_shapes, card_selections,
           leader_rotations, follower_rotations,
           T_count, T_color, T_shape, T_sel, T_lead, T_foll):
    idx_arrays = [a.reshape(_B, 1, _WD) for a in
                  (card_counts, card_colors, card_shapes, card_selections,
                   leader_rotations, follower_rotations)]

    tab = jnp.concatenate([T_count, T_color, T_shape, T_sel, T_lead, T_foll],
                          axis=0)
    row_mask = np.ones((_NROWS, 1), dtype=np.float32)
    for off in _OFFSETS:
        row_mask[off, 0] = 0.0
    tabT = (tab * jnp.asarray(row_mask)).T

    idx_spec = pl.BlockSpec((_B_BLK, 1, _WD), lambda i: (i, 0, 0))
    out = pl.pallas_call(
        _embed_body,
        grid=(_B // _B_BLK,),
        in_specs=[idx_spec] * 6 + [pl.BlockSpec((_E, _NROWS), lambda i: (0, 0))],
        out_specs=pl.BlockSpec((_B_BLK, _E, _WD), lambda i: (i, 0, 0)),
        out_shape=jax.ShapeDtypeStruct((_B, _E, _WD), jnp.float32),
        compiler_params=pltpu.CompilerParams(
            dimension_semantics=("parallel",)),
    )(*idx_arrays, tabT)
    return out.reshape(_B, _E, _W, _D)
